# baseline (speedup 1.0000x reference)
import functools

import jax
import jax.numpy as jnp
from jax import lax
from jax.experimental import pallas as pl
from jax.experimental.pallas import tpu as pltpu
from jax.experimental.pallas import tpu_sc as plsc

B = 16384
E = 32
V = 1000000
NC = 2
NS = 16
NW = NC * NS
BPW = B // NW
CH = 128
NCH = BPW // CH
L = 16


def _sc_body(user_hbm, item_hbm, ie_hbm, fq_hbm, bias_hbm, utab_hbm,
             itab_hbm, ftab_hbm, out_hbm,
             uidx, iidx, iev, fqv, fidx, urowsT, irowsT, fvals, outv, biasv,
             semu, semi, semf):
    wid = lax.axis_index("s") * NC + lax.axis_index("c")
    base = wid * BPW

    pltpu.sync_copy(user_hbm.at[wid], uidx)
    pltpu.sync_copy(item_hbm.at[wid], iidx)
    pltpu.sync_copy(ie_hbm.at[wid], iev)
    pltpu.sync_copy(fq_hbm.at[wid], fqv)
    pltpu.sync_copy(bias_hbm, biasv)

    for k in range(BPW // L):
        s = pl.ds(k * L, L)
        fidx[s] = iev[s] * 1000 + fqv[s]
    for j in range(NCH):
        pltpu.async_copy(ftab_hbm.at[fidx.at[pl.ds(j * CH, CH)]],
                         fvals.at[pl.ds(j * CH, CH)], semf)

    for e in range(E):
        w = pl.ds(e * V, V)
        for j in range(NCH):
            c = pl.ds(j * CH, CH)
            pltpu.async_copy(utab_hbm.at[w].at[uidx.at[c]],
                             urowsT.at[e, c], semu)
            pltpu.async_copy(itab_hbm.at[w].at[iidx.at[c]],
                             irowsT.at[e, c], semi)

    pltpu.make_async_copy(utab_hbm.at[pl.ds(0, E * BPW)],
                          urowsT, semu).wait()
    pltpu.make_async_copy(itab_hbm.at[pl.ds(0, E * BPW)],
                          irowsT, semi).wait()
    pltpu.make_async_copy(ftab_hbm.at[pl.ds(0, BPW)], fvals, semf).wait()

    bias_vec = biasv[pl.ds(0, L)]

    def dot_step(c, carry):
        s = pl.ds(c * L, L)
        a0 = urowsT[0, s] * irowsT[0, s]
        a1 = urowsT[1, s] * irowsT[1, s]
        a2 = urowsT[2, s] * irowsT[2, s]
        a3 = urowsT[3, s] * irowsT[3, s]
        for e in range(4, E):
            p = urowsT[e, s] * irowsT[e, s]
            if e % 4 == 0:
                a0 = a0 + p
            elif e % 4 == 1:
                a1 = a1 + p
            elif e % 4 == 2:
                a2 = a2 + p
            else:
                a3 = a3 + p
        outv[s] = (a0 + a1) + (a2 + a3) + fvals[s] + bias_vec
        return carry

    lax.fori_loop(0, BPW // L, dot_step, 0)

    pltpu.sync_copy(outv, out_hbm.at[pl.ds(base, BPW)])


@jax.jit
def _sfc_forward(user, item, idx_emb, freq, bias, utab, itab, ftab):
    mesh = plsc.VectorSubcoreMesh(core_axis_name="c", subcore_axis_name="s")
    fwd = functools.partial(
        pl.kernel,
        mesh=mesh,
        compiler_params=pltpu.CompilerParams(
            use_tc_tiling_on_sc=False, needs_layout_passes=False),
        out_type=jax.ShapeDtypeStruct((B,), jnp.float32),
        scratch_types=[
            pltpu.VMEM((BPW,), jnp.int32),
            pltpu.VMEM((BPW,), jnp.int32),
            pltpu.VMEM((BPW,), jnp.int32),
            pltpu.VMEM((BPW,), jnp.int32),
            pltpu.VMEM((BPW,), jnp.int32),
            pltpu.VMEM((E, BPW), jnp.float32),
            pltpu.VMEM((E, BPW), jnp.float32),
            pltpu.VMEM((BPW,), jnp.float32),
            pltpu.VMEM((BPW,), jnp.float32),
            pltpu.VMEM((L,), jnp.float32),
            pltpu.SemaphoreType.DMA,
            pltpu.SemaphoreType.DMA,
            pltpu.SemaphoreType.DMA,
        ],
    )(_sc_body)
    return fwd(user, item, idx_emb, freq, bias, utab, itab, ftab)


def kernel(user, item, freq, idx_emb, zero, bias_table, user_table,
           item_table, freq_tables):
    del zero
    user2 = user.astype(jnp.int32).reshape(NW, BPW)
    item2 = item.astype(jnp.int32).reshape(NW, BPW)
    ie2 = idx_emb.astype(jnp.int32).reshape(NW, BPW)
    fq2 = freq.astype(jnp.int32).reshape(NW, BPW)
    bias1 = jnp.broadcast_to(bias_table.reshape(()), (L,))
    ftab_flat = freq_tables.reshape(-1)
    utab_flat = user_table.T.reshape(-1)
    itab_flat = item_table.T.reshape(-1)
    return _sfc_forward(user2, item2, ie2, fq2, bias1, utab_flat,
                        itab_flat, ftab_flat)


# speedup vs baseline: 1.0022x; 1.0022x over previous
import functools

import jax
import jax.numpy as jnp
from jax import lax
from jax.experimental import pallas as pl
from jax.experimental.pallas import tpu as pltpu
from jax.experimental.pallas import tpu_sc as plsc

B = 16384
E = 32
V = 1000000
NC = 2
NS = 16
NW = NC * NS
BPW = B // NW
CH = 128
NCH = BPW // CH
L = 16


def _sc_body(user_hbm, item_hbm, ie_hbm, fq_hbm, bias_hbm, utab_hbm,
             itab_hbm, ftab_hbm, out_hbm,
             uidx, iidx, iev, fqv, fidx, urowsT, irowsT, fvals, outv, biasv,
             semu, semi, semf):
    wid = lax.axis_index("s") * NC + lax.axis_index("c")
    base = wid * BPW

    pltpu.sync_copy(user_hbm.at[wid], uidx)
    pltpu.sync_copy(item_hbm.at[wid], iidx)
    pltpu.sync_copy(ie_hbm.at[wid], iev)
    pltpu.sync_copy(fq_hbm.at[wid], fqv)
    pltpu.sync_copy(bias_hbm, biasv)

    for k in range(BPW // L):
        s = pl.ds(k * L, L)
        fidx[s] = iev[s] * 1000 + fqv[s]
    for j in range(NCH):
        pltpu.async_copy(ftab_hbm.at[fidx.at[pl.ds(j * CH, CH)]],
                         fvals.at[pl.ds(j * CH, CH)], semf)

    for e in range(E):
        for j in range(NCH):
            c = pl.ds(j * CH, CH)
            pltpu.async_copy(utab_hbm.at[e].at[uidx.at[c]],
                             urowsT.at[e, c], semu)
            pltpu.async_copy(itab_hbm.at[e].at[iidx.at[c]],
                             irowsT.at[e, c], semi)

    pltpu.make_async_copy(utab_hbm.at[:, pl.ds(0, BPW)], urowsT,
                          semu).wait()
    pltpu.make_async_copy(itab_hbm.at[:, pl.ds(0, BPW)], irowsT,
                          semi).wait()
    pltpu.make_async_copy(ftab_hbm.at[pl.ds(0, BPW)], fvals, semf).wait()

    bias_vec = biasv[pl.ds(0, L)]

    def dot_step(c, carry):
        s = pl.ds(c * L, L)
        a0 = urowsT[0, s] * irowsT[0, s]
        a1 = urowsT[1, s] * irowsT[1, s]
        a2 = urowsT[2, s] * irowsT[2, s]
        a3 = urowsT[3, s] * irowsT[3, s]
        for e in range(4, E):
            p = urowsT[e, s] * irowsT[e, s]
            if e % 4 == 0:
                a0 = a0 + p
            elif e % 4 == 1:
                a1 = a1 + p
            elif e % 4 == 2:
                a2 = a2 + p
            else:
                a3 = a3 + p
        outv[s] = (a0 + a1) + (a2 + a3) + fvals[s] + bias_vec
        return carry

    lax.fori_loop(0, BPW // L, dot_step, 0)

    pltpu.sync_copy(outv, out_hbm.at[pl.ds(base, BPW)])


@jax.jit
def _sfc_forward(user, item, idx_emb, freq, bias, utab, itab, ftab):
    mesh = plsc.VectorSubcoreMesh(core_axis_name="c", subcore_axis_name="s")
    fwd = functools.partial(
        pl.kernel,
        mesh=mesh,
        compiler_params=pltpu.CompilerParams(
            use_tc_tiling_on_sc=False, needs_layout_passes=False),
        out_type=jax.ShapeDtypeStruct((B,), jnp.float32),
        scratch_types=[
            pltpu.VMEM((BPW,), jnp.int32),
            pltpu.VMEM((BPW,), jnp.int32),
            pltpu.VMEM((BPW,), jnp.int32),
            pltpu.VMEM((BPW,), jnp.int32),
            pltpu.VMEM((BPW,), jnp.int32),
            pltpu.VMEM((E, BPW), jnp.float32),
            pltpu.VMEM((E, BPW), jnp.float32),
            pltpu.VMEM((BPW,), jnp.float32),
            pltpu.VMEM((BPW,), jnp.float32),
            pltpu.VMEM((L,), jnp.float32),
            pltpu.SemaphoreType.DMA,
            pltpu.SemaphoreType.DMA,
            pltpu.SemaphoreType.DMA,
        ],
    )(_sc_body)
    return fwd(user, item, idx_emb, freq, bias, utab, itab, ftab)


def kernel(user, item, freq, idx_emb, zero, bias_table, user_table,
           item_table, freq_tables):
    del zero
    user2 = user.astype(jnp.int32).reshape(NW, BPW)
    item2 = item.astype(jnp.int32).reshape(NW, BPW)
    ie2 = idx_emb.astype(jnp.int32).reshape(NW, BPW)
    fq2 = freq.astype(jnp.int32).reshape(NW, BPW)
    bias1 = jnp.broadcast_to(bias_table.reshape(()), (L,))
    ftab_flat = freq_tables.reshape(-1)
    return _sfc_forward(user2, item2, ie2, fq2, bias1, user_table.T,
                        item_table.T, ftab_flat)


# speedup vs baseline: 5.4151x; 5.4035x over previous
import functools

import jax
import jax.numpy as jnp
from jax import lax
from jax.experimental import pallas as pl
from jax.experimental.pallas import tpu as pltpu
from jax.experimental.pallas import tpu_sc as plsc

B = 16384
E = 32
NC = 2
NS = 16
NW = NC * NS
BPW = B // NW
CH = 128
NCH = BPW // CH
L = 16


def _sc_body(user_hbm, item_hbm, ie_hbm, fq_hbm, bias_hbm, utab_hbm,
             itab_hbm, ftab_hbm, out_hbm,
             uidx, iidx, iev, fqv, fidx, uix, iix, urowsT, irowsT, fvals,
             outv, biasv, semu, semi, semf):
    wid = lax.axis_index("s") * NC + lax.axis_index("c")
    base = wid * BPW

    pltpu.sync_copy(user_hbm.at[wid], uidx)
    pltpu.sync_copy(item_hbm.at[wid], iidx)
    pltpu.sync_copy(ie_hbm.at[wid], iev)
    pltpu.sync_copy(fq_hbm.at[wid], fqv)
    pltpu.sync_copy(bias_hbm, biasv)

    for j in range(NCH):
        for k in range(CH // L):
            s = pl.ds(k * L, L)
            fidx[j, s] = iev[j, s] * 1000 + fqv[j, s]
    for j in range(NCH):
        pltpu.async_copy(ftab_hbm.at[fidx.at[j]],
                         fvals.at[pl.ds(j * CH, CH)], semf)

    for j in range(NCH):
        for k in range(CH // L):
            s = pl.ds(k * L, L)
            uidx[j, s] = uidx[j, s] * E
            iidx[j, s] = iidx[j, s] * E

    def fire(e, carry):
        for j in range(NCH):
            for k in range(CH // L):
                s = pl.ds(k * L, L)
                uix[j, s] = uidx[j, s] + e
                iix[j, s] = iidx[j, s] + e
        for j in range(NCH):
            c = pl.ds(j * CH, CH)
            pltpu.async_copy(utab_hbm.at[uix.at[j]], urowsT.at[e, c], semu)
            pltpu.async_copy(itab_hbm.at[iix.at[j]], irowsT.at[e, c], semi)
        pltpu.make_async_copy(utab_hbm.at[pl.ds(0, BPW)],
                              urowsT.at[e], semu).wait()
        pltpu.make_async_copy(itab_hbm.at[pl.ds(0, BPW)],
                              irowsT.at[e], semi).wait()
        return carry

    lax.fori_loop(0, E, fire, 0)
    pltpu.make_async_copy(ftab_hbm.at[pl.ds(0, BPW)], fvals, semf).wait()

    bias_vec = biasv[pl.ds(0, L)]

    def dot_step(c, carry):
        s = pl.ds(c * L, L)
        a0 = urowsT[0, s] * irowsT[0, s]
        a1 = urowsT[1, s] * irowsT[1, s]
        a2 = urowsT[2, s] * irowsT[2, s]
        a3 = urowsT[3, s] * irowsT[3, s]
        for e in range(4, E):
            p = urowsT[e, s] * irowsT[e, s]
            if e % 4 == 0:
                a0 = a0 + p
            elif e % 4 == 1:
                a1 = a1 + p
            elif e % 4 == 2:
                a2 = a2 + p
            else:
                a3 = a3 + p
        outv[s] = (a0 + a1) + (a2 + a3) + fvals[s] + bias_vec
        return carry

    lax.fori_loop(0, BPW // L, dot_step, 0)

    pltpu.sync_copy(outv, out_hbm.at[pl.ds(base, BPW)])


@jax.jit
def _sfc_forward(user, item, idx_emb, freq, bias, utab, itab, ftab):
    mesh = plsc.VectorSubcoreMesh(core_axis_name="c", subcore_axis_name="s")
    fwd = functools.partial(
        pl.kernel,
        mesh=mesh,
        compiler_params=pltpu.CompilerParams(
            use_tc_tiling_on_sc=False, needs_layout_passes=False),
        out_type=jax.ShapeDtypeStruct((B,), jnp.float32),
        scratch_types=[
            pltpu.VMEM((NCH, CH), jnp.int32),
            pltpu.VMEM((NCH, CH), jnp.int32),
            pltpu.VMEM((NCH, CH), jnp.int32),
            pltpu.VMEM((NCH, CH), jnp.int32),
            pltpu.VMEM((NCH, CH), jnp.int32),
            pltpu.VMEM((NCH, CH), jnp.int32),
            pltpu.VMEM((NCH, CH), jnp.int32),
            pltpu.VMEM((E, BPW), jnp.float32),
            pltpu.VMEM((E, BPW), jnp.float32),
            pltpu.VMEM((BPW,), jnp.float32),
            pltpu.VMEM((BPW,), jnp.float32),
            pltpu.VMEM((L,), jnp.float32),
            pltpu.SemaphoreType.DMA,
            pltpu.SemaphoreType.DMA,
            pltpu.SemaphoreType.DMA,
        ],
    )(_sc_body)
    return fwd(user, item, idx_emb, freq, bias, utab, itab, ftab)


def kernel(user, item, freq, idx_emb, zero, bias_table, user_table,
           item_table, freq_tables):
    del zero
    user3 = user.astype(jnp.int32).reshape(NW, NCH, CH)
    item3 = item.astype(jnp.int32).reshape(NW, NCH, CH)
    ie3 = idx_emb.astype(jnp.int32).reshape(NW, NCH, CH)
    fq3 = freq.astype(jnp.int32).reshape(NW, NCH, CH)
    bias1 = jnp.broadcast_to(bias_table.reshape(()), (L,))
    ftab_flat = freq_tables.reshape(-1)
    return _sfc_forward(user3, item3, ie3, fq3, bias1,
                        user_table.reshape(-1), item_table.reshape(-1),
                        ftab_flat)


# speedup vs baseline: 12.5649x; 2.3203x over previous
import functools

import jax
import jax.numpy as jnp
from jax import lax
from jax.experimental import pallas as pl
from jax.experimental.pallas import tpu as pltpu
from jax.experimental.pallas import tpu_sc as plsc

B = 16384
E = 32
V = 1000000
NC = 2
NS = 16
NW = NC * NS
BPW = B // NW
CK = 32
NCK = BPW // CK
L = 16
FPAD = 26112


def _sc_body(user_hbm, item_hbm, ie_hbm, fq_hbm, bias_hbm, utab_hbm,
             itab_hbm, ftab_hbm, out_hbm,
             uidx, iidx, iev, fqv, outv, biasv, gu, gv, gf,
             sem, semf):
    wid = lax.axis_index("s") * NC + lax.axis_index("c")
    base = wid * BPW

    pltpu.sync_copy(user_hbm.at[wid], uidx)
    pltpu.sync_copy(item_hbm.at[wid], iidx)
    pltpu.sync_copy(ie_hbm.at[wid], iev)
    pltpu.sync_copy(fq_hbm.at[wid], fqv)
    pltpu.sync_copy(bias_hbm, biasv)

    iota = lax.broadcasted_iota(jnp.int32, (L,), 0)
    cols = [jnp.full((L,), e, dtype=jnp.int32) for e in range(E)]
    bias_vec = biasv[pl.ds(0, L)]
    seven = jnp.full((L,), 7, dtype=jnp.int32)
    m127 = jnp.full((L,), 127, dtype=jnp.int32)

    def chunk(c, carry):
        r0 = c * CK
        copies = []
        for k in range(CK // L):
            s = pl.ds(r0 + k * L, L)
            tu16 = lax.shift_right_logical(uidx[s], 3)
            tv16 = lax.shift_right_logical(iidx[s], 3)
            for i in range(L):
                r = k * L + i
                copies.append(pltpu.async_copy(utab_hbm.at[tu16[i]],
                                               gu.at[r], sem))
                copies.append(pltpu.async_copy(itab_hbm.at[tv16[i]],
                                               gv.at[r], sem))
        for k in range(CK // L):
            s = pl.ds(r0 + k * L, L)
            f16 = iev[s] * 1000 + fqv[s]
            tf = lax.shift_right_logical(f16, 7)
            copies.append(pltpu.async_copy(ftab_hbm.at[tf],
                                           gf.at[pl.ds(k * L, L)], semf))
        for cp in copies:
            cp.wait()

        for k in range(CK // L):
            s = pl.ds(r0 + k * L, L)
            su = lax.bitwise_and(uidx[s], seven)
            sv = lax.bitwise_and(iidx[s], seven)
            f16 = iev[s] * 1000 + fqv[s]
            sf = lax.bitwise_and(f16, m127)
            row16 = iota + k * L
            fv = plsc.load_gather(gf, [row16, sf])
            acc = [fv + bias_vec, None, None, None]
            for e in range(E):
                ue = plsc.load_gather(gu, [row16, su, cols[e]])
                ve = plsc.load_gather(gv, [row16, sv, cols[e]])
                p = ue * ve
                a = e % 4
                acc[a] = p if acc[a] is None else acc[a] + p
            outv[s] = (acc[0] + acc[1]) + (acc[2] + acc[3])
        return carry

    lax.fori_loop(0, NCK, chunk, 0)

    pltpu.sync_copy(outv, out_hbm.at[pl.ds(base, BPW)])


@jax.jit
def _sfc_forward(user, item, idx_emb, freq, bias, utab, itab, ftab):
    mesh = plsc.VectorSubcoreMesh(core_axis_name="c", subcore_axis_name="s")
    fwd = functools.partial(
        pl.kernel,
        mesh=mesh,
        compiler_params=pltpu.CompilerParams(needs_layout_passes=False),
        out_type=jax.ShapeDtypeStruct((B,), jnp.float32),
        scratch_types=[
            pltpu.VMEM((BPW,), jnp.int32),
            pltpu.VMEM((BPW,), jnp.int32),
            pltpu.VMEM((BPW,), jnp.int32),
            pltpu.VMEM((BPW,), jnp.int32),
            pltpu.VMEM((BPW,), jnp.float32),
            pltpu.VMEM((L,), jnp.float32),
            pltpu.VMEM((CK, 8, E), jnp.float32),
            pltpu.VMEM((CK, 8, E), jnp.float32),
            pltpu.VMEM((CK, 128), jnp.float32),
            pltpu.SemaphoreType.DMA,
            pltpu.SemaphoreType.DMA,
        ],
    )(_sc_body)
    return fwd(user, item, idx_emb, freq, bias, utab, itab, ftab)


def kernel(user, item, freq, idx_emb, zero, bias_table, user_table,
           item_table, freq_tables):
    del zero
    user2 = user.astype(jnp.int32).reshape(NW, BPW)
    item2 = item.astype(jnp.int32).reshape(NW, BPW)
    ie2 = idx_emb.astype(jnp.int32).reshape(NW, BPW)
    fq2 = freq.astype(jnp.int32).reshape(NW, BPW)
    bias1 = jnp.broadcast_to(bias_table.reshape(()), (L,))
    ftab2 = jnp.pad(freq_tables.reshape(-1),
                    (0, FPAD - 26000)).reshape(FPAD // 128, 128)
    ut3 = user_table.reshape(V // 8, 8, E)
    it3 = item_table.reshape(V // 8, 8, E)
    return _sfc_forward(user2, item2, ie2, fq2, bias1, ut3, it3, ftab2)
